# baseline (speedup 1.0000x reference)
import functools
import math

import jax
import jax.numpy as jnp
from jax import lax
from jax.experimental import pallas as pl
from jax.experimental.pallas import tpu as pltpu
from jax.experimental.pallas import tpu_sc as plsc

EMB = 64
SCALE = math.sqrt(EMB)

NC = 2
NS = 16
NW = NC * NS

RPC = 2
NBUF = 3


def _emb_body(rows_per_tile, seq, tokens_hbm, table_hbm, out_hbm,
              idx_v, rows_v, g_sems, s_sems):
    wid = lax.axis_index("s") * NC + lax.axis_index("c")
    base = wid * rows_per_tile
    n_chunks = rows_per_tile // RPC
    segs = []
    off = 0
    while off < seq:
        g = min(128, seq - off)
        segs.append((off, g))
        off += g

    pltpu.sync_copy(tokens_hbm.at[pl.ds(base, rows_per_tile)], idx_v)

    def start_gathers(c, buf):
        for r in range(RPC):
            for (o, g) in segs:
                pltpu.async_copy(
                    table_hbm.at[idx_v.at[c * RPC + r, pl.ds(o, g)]],
                    rows_v.at[buf, r, pl.ds(o, g)],
                    g_sems.at[buf],
                )

    def drain_gathers(c, buf):
        for r in range(RPC):
            for (o, g) in segs:
                pltpu.make_async_copy(
                    table_hbm.at[idx_v.at[c * RPC + r, pl.ds(o, g)]],
                    rows_v.at[buf, r, pl.ds(o, g)],
                    g_sems.at[buf],
                ).wait()

    def wait_store(c, buf):
        pltpu.make_async_copy(
            rows_v.at[buf],
            out_hbm.at[pl.ds(base + c * RPC, RPC)],
            s_sems.at[buf],
        ).wait()

    start_gathers(0, 0)
    start_gathers(1, 1)

    def chunk_body(c, carry):
        buf = lax.rem(c, NBUF)
        drain_gathers(c, buf)

        @plsc.parallel_loop(0, seq, unroll=4)
        def scale_row(r):
            vals = [
                rows_v[buf, rr, r, pl.ds(j * 16, 16)]
                for rr in range(RPC)
                for j in range(EMB // 16)
            ]
            k = 0
            for rr in range(RPC):
                for j in range(EMB // 16):
                    rows_v[buf, rr, r, pl.ds(j * 16, 16)] = vals[k] * SCALE
                    k += 1

        pltpu.async_copy(
            rows_v.at[buf],
            out_hbm.at[pl.ds(base + c * RPC, RPC)],
            s_sems.at[buf],
        )

        @pl.when(c + 2 < n_chunks)
        def _():
            nb = lax.rem(c + 2, NBUF)

            @pl.when(c >= 1)
            def _():
                wait_store(c - 1, nb)

            start_gathers(c + 2, nb)

        return carry

    lax.fori_loop(0, n_chunks, chunk_body, 0)

    for t in (3, 2, 1):
        c = n_chunks - t
        wait_store(c, lax.rem(c, NBUF))


def kernel(tokens, embedding_weight):
    b, s = tokens.shape
    assert b % (NW * RPC) == 0
    rows_per_tile = b // NW

    mesh = plsc.VectorSubcoreMesh(core_axis_name="c", subcore_axis_name="s")
    run = pl.kernel(
        functools.partial(_emb_body, rows_per_tile, s),
        mesh=mesh,
        out_type=jax.ShapeDtypeStruct((b, s, EMB), jnp.float32),
        scratch_types=[
            pltpu.VMEM((rows_per_tile, s), jnp.int32),
            pltpu.VMEM((NBUF, RPC, s, EMB), jnp.float32),
            pltpu.SemaphoreType.DMA((NBUF,)),
            pltpu.SemaphoreType.DMA((NBUF,)),
        ],
        compiler_params=pltpu.CompilerParams(use_tc_tiling_on_sc=False),
    )
    return run(tokens.astype(jnp.int32), embedding_weight)
